# baseline (speedup 1.0000x reference)
import jax
import jax.numpy as jnp
from jax import lax
from jax.experimental import pallas as pl
from jax.experimental.pallas import tpu as pltpu

_H = 24
_L = _H * _H
_MASK_RATIO = 0.6
_LEN_KEEP = int(_L * (1.0 - _MASK_RATIO))
_ROWS = 8


def _mask_body(n_ref, out_ref):
    n = n_ref[...]
    b = lax.bitcast_convert_type(n, jnp.int32)
    Bn = b.shape[0]
    k0 = jnp.full((Bn, 1), _LEN_KEEP, jnp.int32)

    def descent(bits, nbits, valid, k_init):
        def step(i, carry):
            prefix, k = carry
            bit = nbits - 1 - i
            hi = lax.shift_right_logical(bits, bit + 1)
            phi = lax.shift_right_logical(prefix, bit + 1)
            b0 = lax.shift_right_logical(bits, bit) & 1
            sel = valid & (hi == phi) & (b0 == 0)
            c = jnp.sum(sel.astype(jnp.int32), axis=1, keepdims=True)
            take1 = k > c
            k = jnp.where(take1, k - c, k)
            prefix = jnp.where(take1, prefix | (1 << bit), prefix)
            return prefix, k
        prefix, _ = lax.fori_loop(0, nbits, step,
                                  (jnp.zeros((Bn, 1), jnp.int32), k_init))
        return prefix

    t = descent(b, 30, jnp.full(b.shape, True), k0)
    lt = b < t
    cnt_lt = jnp.sum(lt.astype(jnp.int32), axis=1, keepdims=True)
    eq = b == t
    need = k0 - cnt_lt
    idx = lax.broadcasted_iota(jnp.int32, b.shape, 1)
    it = descent(idx, 10, eq, need)
    keep = lt | (eq & (idx <= it))
    out_ref[...] = keep.astype(jnp.float32)


def _expand(k, m):
    s = _H * k
    a0 = lax.broadcasted_iota(jnp.int32, (s, _H), 0)
    a1 = lax.broadcasted_iota(jnp.int32, (s, _H), 1)
    A = (a0 // k == a1).astype(jnp.float32)
    b0 = lax.broadcasted_iota(jnp.int32, (_H, s), 0)
    b1 = lax.broadcasted_iota(jnp.int32, (_H, s), 1)
    Bt = (b0 == b1 // k).astype(jnp.float32)
    t = jnp.dot(A, m, preferred_element_type=jnp.float32)
    return jnp.dot(t, Bt, preferred_element_type=jnp.float32)


_AB = 4


def _apply_body(m_ref, x_ref, y_ref, o24_ref, o48_ref, o96_ref,
                o192_ref, o384_ref):
    for b in range(_AB):
        m24 = m_ref[b]
        m48 = _expand(2, m24)
        m96 = _expand(4, m24)
        m192 = _expand(8, m24)
        m384 = _expand(16, m24)
        o24_ref[b, 0] = m24 > 0.5
        o48_ref[b, 0] = m48 > 0.5
        o96_ref[b, 0] = m96 > 0.5
        o192_ref[b, 0] = m192 > 0.5
        o384_ref[b, 0] = m384 > 0.5
        y_ref[b] = x_ref[b] * m384[None]


def kernel(inp_bchw):
    B, C, Hh, Ww = inp_bchw.shape
    noise = jax.random.uniform(jax.random.key(42), (B, _L), dtype=jnp.float32)

    mask_flat = pl.pallas_call(
        _mask_body,
        out_shape=jax.ShapeDtypeStruct((B, _L), jnp.float32),
    )(noise)

    m2d = mask_flat.reshape(B, _H, _H)

    out_shapes = (
        jax.ShapeDtypeStruct((B, C, Hh, Ww), jnp.float32),
        jax.ShapeDtypeStruct((B, 1, _H, _H), jnp.bool_),
        jax.ShapeDtypeStruct((B, 1, 2 * _H, 2 * _H), jnp.bool_),
        jax.ShapeDtypeStruct((B, 1, 4 * _H, 4 * _H), jnp.bool_),
        jax.ShapeDtypeStruct((B, 1, 8 * _H, 8 * _H), jnp.bool_),
        jax.ShapeDtypeStruct((B, 1, 16 * _H, 16 * _H), jnp.bool_),
    )
    lvl_spec = lambda s: pl.BlockSpec((_AB, 1, s, s), lambda b: (b, 0, 0, 0))
    masked, l24, l48, l96, l192, l384 = pl.pallas_call(
        _apply_body,
        grid=(B // _AB,),
        in_specs=[
            pl.BlockSpec((_AB, _H, _H), lambda b: (b, 0, 0)),
            pl.BlockSpec((_AB, C, Hh, Ww), lambda b: (b, 0, 0, 0)),
        ],
        out_specs=[
            pl.BlockSpec((_AB, C, Hh, Ww), lambda b: (b, 0, 0, 0)),
            lvl_spec(_H), lvl_spec(2 * _H), lvl_spec(4 * _H),
            lvl_spec(8 * _H), lvl_spec(16 * _H),
        ],
        out_shape=out_shapes,
        compiler_params=pltpu.CompilerParams(
            dimension_semantics=("parallel",)),
    )(m2d, inp_bchw)

    return (masked, l24, l48, l96, l192, l384)
